# baseline (speedup 1.0000x reference)
import functools

import jax
import jax.numpy as jnp
from jax import lax
from jax.experimental import pallas as pl
from jax.experimental.pallas import tpu as pltpu
from jax.experimental.pallas import tpu_sc as plsc

HIDDEN = 1024
EPS = 1e-12

_NC = 2
_NS = 16
_NW = _NC * _NS


def _make_sc_gather(n_tok: int, ch: int):
    n_per_w = n_tok // _NW
    nch = n_per_w // ch
    mesh = plsc.VectorSubcoreMesh(core_axis_name="c", subcore_axis_name="s")

    @functools.partial(
        pl.kernel,
        mesh=mesh,
        out_type=jax.ShapeDtypeStruct((n_tok, HIDDEN), jnp.float32),
        scratch_types=[
            pltpu.VMEM((n_per_w,), jnp.int32),
            pltpu.VMEM((ch, HIDDEN), jnp.float32),
            pltpu.VMEM((ch, HIDDEN), jnp.float32),
            pltpu.SemaphoreType.DMA,
            pltpu.SemaphoreType.DMA,
            pltpu.SemaphoreType.DMA,
        ],
    )
    def gather_k(ids_hbm, table_hbm, out_hbm, ids_v, r0, r1, gsem, w0sem, w1sem):
        wid = lax.axis_index("s") * _NC + lax.axis_index("c")
        base = wid * n_per_w
        pltpu.sync_copy(ids_hbm.at[pl.ds(base, n_per_w)], ids_v)

        bufs = (r0, r1)
        wsems = (w0sem, w1sem)

        def start_gather(c, buf):
            return pltpu.async_copy(
                table_hbm.at[ids_v.at[pl.ds(c * ch, ch)]], buf, gsem
            )

        pending = [None, None]
        g_prev = start_gather(0, r0)
        for c in range(nch):
            b = c % 2
            g_prev.wait()
            if c + 1 < nch:
                nb = (c + 1) % 2
                if pending[nb] is not None:
                    pending[nb].wait()
                    pending[nb] = None
                g_prev = start_gather(c + 1, bufs[nb])
            pending[b] = pltpu.async_copy(
                bufs[b], out_hbm.at[pl.ds(base + c * ch, ch)], wsems[b]
            )
        for b in range(2):
            if pending[b] is not None:
                pending[b].wait()

    return gather_k


def _make_tc_ln(n_tok: int, seq_len: int, batch: int, blk: int):
    pos_blocks = seq_len // blk

    def body(x_ref, pos_ref, type_ref, g_ref, b_ref, o_ref):
        x = x_ref[...] + pos_ref[...] + type_ref[0:1, :]
        mu = jnp.mean(x, axis=-1, keepdims=True)
        xc = x - mu
        var = jnp.mean(xc * xc, axis=-1, keepdims=True)
        rstd = lax.rsqrt(var + EPS)
        o_ref[...] = (xc * rstd) * g_ref[0:1, :] + b_ref[0:1, :]

    return pl.pallas_call(
        body,
        grid=(pos_blocks, batch),
        in_specs=[
            pl.BlockSpec((blk, HIDDEN), lambda p, b: (b * pos_blocks + p, 0)),
            pl.BlockSpec((blk, HIDDEN), lambda p, b: (p, 0)),
            pl.BlockSpec((2, HIDDEN), lambda p, b: (0, 0)),
            pl.BlockSpec((1, HIDDEN), lambda p, b: (0, 0)),
            pl.BlockSpec((1, HIDDEN), lambda p, b: (0, 0)),
        ],
        out_specs=pl.BlockSpec((blk, HIDDEN), lambda p, b: (b * pos_blocks + p, 0)),
        out_shape=jax.ShapeDtypeStruct((n_tok, HIDDEN), jnp.float32),
    )


def kernel(input_ids, word_table, pos_table, type_table, ln_gamma, ln_beta):
    B, L = input_ids.shape
    ids = input_ids.astype(jnp.int32)
    gamma2 = ln_gamma.reshape(1, HIDDEN)
    beta2 = ln_beta.reshape(1, HIDDEN)
    sc_gather = _make_sc_gather(L, ch=32)
    tc_ln = _make_tc_ln(L, L, 1, blk=512)
    outs = []
    for b in range(B):
        rows = sc_gather(ids[b], word_table)
        outs.append(tc_ln(rows, pos_table, type_table, gamma2, beta2))
    return jnp.stack(outs, axis=0)


# speedup vs baseline: 1.4533x; 1.4533x over previous
import functools

import jax
import jax.numpy as jnp
from jax import lax
from jax.experimental import pallas as pl
from jax.experimental.pallas import tpu as pltpu
from jax.experimental.pallas import tpu_sc as plsc

HIDDEN = 1024
EPS = 1e-12

_NC = 2
_NS = 16
_NW = _NC * _NS


def _make_sc_gather(n_tok: int, ch: int):
    n_per_w = n_tok // _NW
    nch = n_per_w // ch
    mesh = plsc.VectorSubcoreMesh(core_axis_name="c", subcore_axis_name="s")

    @functools.partial(
        pl.kernel,
        mesh=mesh,
        out_type=jax.ShapeDtypeStruct((n_tok, HIDDEN), jnp.float32),
        scratch_types=[
            pltpu.VMEM((n_per_w,), jnp.int32),
            pltpu.VMEM((ch, HIDDEN), jnp.float32),
            pltpu.VMEM((ch, HIDDEN), jnp.float32),
            pltpu.SemaphoreType.DMA,
            pltpu.SemaphoreType.DMA,
            pltpu.SemaphoreType.DMA,
        ],
    )
    def gather_k(ids_hbm, table_hbm, out_hbm, ids_v, r0, r1, gsem, w0sem, w1sem):
        wid = lax.axis_index("s") * _NC + lax.axis_index("c")
        base = wid * n_per_w
        pltpu.sync_copy(ids_hbm.at[pl.ds(base, n_per_w)], ids_v)

        bufs = (r0, r1)
        wsems = (w0sem, w1sem)

        def start_gather(c, buf):
            return pltpu.async_copy(
                table_hbm.at[ids_v.at[pl.ds(c * ch, ch)]], buf, gsem
            )

        pending = [None, None]
        g_prev = start_gather(0, r0)
        for c in range(nch):
            b = c % 2
            g_prev.wait()
            if c + 1 < nch:
                nb = (c + 1) % 2
                if pending[nb] is not None:
                    pending[nb].wait()
                    pending[nb] = None
                g_prev = start_gather(c + 1, bufs[nb])
            pending[b] = pltpu.async_copy(
                bufs[b], out_hbm.at[pl.ds(base + c * ch, ch)], wsems[b]
            )
        for b in range(2):
            if pending[b] is not None:
                pending[b].wait()

    return gather_k


def _make_tc_ln(n_tok: int, seq_len: int, batch: int, blk: int):
    pos_blocks = seq_len // blk

    def body(x_ref, pos_ref, type_ref, g_ref, b_ref, o_ref):
        x = x_ref[...] + pos_ref[...] + type_ref[0:1, :]
        mu = jnp.mean(x, axis=-1, keepdims=True)
        xc = x - mu
        var = jnp.mean(xc * xc, axis=-1, keepdims=True)
        rstd = lax.rsqrt(var + EPS)
        o_ref[...] = (xc * rstd) * g_ref[0:1, :] + b_ref[0:1, :]

    return pl.pallas_call(
        body,
        grid=(pos_blocks, batch),
        in_specs=[
            pl.BlockSpec((blk, HIDDEN), lambda p, b: (b * pos_blocks + p, 0)),
            pl.BlockSpec((blk, HIDDEN), lambda p, b: (p, 0)),
            pl.BlockSpec((2, HIDDEN), lambda p, b: (0, 0)),
            pl.BlockSpec((1, HIDDEN), lambda p, b: (0, 0)),
            pl.BlockSpec((1, HIDDEN), lambda p, b: (0, 0)),
        ],
        out_specs=pl.BlockSpec((blk, HIDDEN), lambda p, b: (b * pos_blocks + p, 0)),
        out_shape=jax.ShapeDtypeStruct((n_tok, HIDDEN), jnp.float32),
    )


def kernel(input_ids, word_table, pos_table, type_table, ln_gamma, ln_beta):
    B, L = input_ids.shape
    n_tok = B * L
    ids = input_ids.reshape(n_tok).astype(jnp.int32)
    rows = _make_sc_gather(n_tok, ch=32)(ids, word_table)
    out = _make_tc_ln(n_tok, L, B, blk=1024)(
        rows,
        pos_table,
        type_table,
        ln_gamma.reshape(1, HIDDEN),
        ln_beta.reshape(1, HIDDEN),
    )
    return out.reshape(B, L, HIDDEN)


# speedup vs baseline: 1.4627x; 1.0065x over previous
import functools

import jax
import jax.numpy as jnp
from jax import lax
from jax.experimental import pallas as pl
from jax.experimental.pallas import tpu as pltpu
from jax.experimental.pallas import tpu_sc as plsc

HIDDEN = 1024
EPS = 1e-12

_NC = 2
_NS = 16
_NW = _NC * _NS


def _make_sc_gather(n_tok: int, ch: int):
    n_per_w = n_tok // _NW
    nch = n_per_w // ch
    mesh = plsc.VectorSubcoreMesh(core_axis_name="c", subcore_axis_name="s")

    @functools.partial(
        pl.kernel,
        mesh=mesh,
        out_type=jax.ShapeDtypeStruct((n_tok, HIDDEN), jnp.float32),
        scratch_types=[
            pltpu.VMEM((n_per_w,), jnp.int32),
            pltpu.VMEM((ch, HIDDEN), jnp.float32),
            pltpu.VMEM((ch, HIDDEN), jnp.float32),
            pltpu.SemaphoreType.DMA,
            pltpu.SemaphoreType.DMA,
            pltpu.SemaphoreType.DMA,
        ],
    )
    def gather_k(ids_hbm, table_hbm, out_hbm, ids_v, r0, r1, gsem, w0sem, w1sem):
        wid = lax.axis_index("s") * _NC + lax.axis_index("c")
        base = wid * n_per_w
        pltpu.sync_copy(ids_hbm.at[pl.ds(base, n_per_w)], ids_v)

        bufs = (r0, r1)
        wsems = (w0sem, w1sem)

        def start_gather(c, buf):
            return pltpu.async_copy(
                table_hbm.at[ids_v.at[pl.ds(c * ch, ch)]], buf, gsem
            )

        pending = [None, None]
        g_prev = start_gather(0, r0)
        for c in range(nch):
            b = c % 2
            g_prev.wait()
            if c + 1 < nch:
                nb = (c + 1) % 2
                if pending[nb] is not None:
                    pending[nb].wait()
                    pending[nb] = None
                g_prev = start_gather(c + 1, bufs[nb])
            pending[b] = pltpu.async_copy(
                bufs[b], out_hbm.at[pl.ds(base + c * ch, ch)], wsems[b]
            )
        for b in range(2):
            if pending[b] is not None:
                pending[b].wait()

    return gather_k


def _make_tc_ln(n_tok: int, seq_len: int, batch: int, blk: int):
    pos_blocks = seq_len // blk

    def body(x_ref, pos_ref, type_ref, g_ref, b_ref, o_ref):
        x = x_ref[...] + pos_ref[...] + type_ref[0:1, :]
        mu = jnp.mean(x, axis=-1, keepdims=True)
        xc = x - mu
        var = jnp.mean(xc * xc, axis=-1, keepdims=True)
        rstd = lax.rsqrt(var + EPS)
        o_ref[...] = (xc * rstd) * g_ref[0:1, :] + b_ref[0:1, :]

    return pl.pallas_call(
        body,
        grid=(pos_blocks, batch),
        in_specs=[
            pl.BlockSpec((blk, HIDDEN), lambda p, b: (b * pos_blocks + p, 0)),
            pl.BlockSpec((blk, HIDDEN), lambda p, b: (p, 0)),
            pl.BlockSpec((2, HIDDEN), lambda p, b: (0, 0)),
            pl.BlockSpec((1, HIDDEN), lambda p, b: (0, 0)),
            pl.BlockSpec((1, HIDDEN), lambda p, b: (0, 0)),
        ],
        out_specs=pl.BlockSpec((blk, HIDDEN), lambda p, b: (b * pos_blocks + p, 0)),
        out_shape=jax.ShapeDtypeStruct((n_tok, HIDDEN), jnp.float32),
    )


def kernel(input_ids, word_table, pos_table, type_table, ln_gamma, ln_beta):
    B, L = input_ids.shape
    n_tok = B * L
    ids = input_ids.reshape(n_tok).astype(jnp.int32)
    rows = _make_sc_gather(n_tok, ch=32)(ids, word_table)
    out = _make_tc_ln(n_tok, L, B, blk=2048)(
        rows,
        pos_table,
        type_table,
        ln_gamma.reshape(1, HIDDEN),
        ln_beta.reshape(1, HIDDEN),
    )
    return out.reshape(B, L, HIDDEN)
